# baseline (speedup 1.0000x reference)
import functools

import jax
import jax.numpy as jnp
from jax import lax
from jax.experimental import pallas as pl
from jax.experimental.pallas import tpu as pltpu
from jax.experimental.pallas import tpu_sc as plsc

N = 10000
E = 320000
D = 128
EIG_IDX = 1
ROWS = 1000
GRID = N // ROWS

NC = 2
NS = 16
NW = NC * NS
DPT = 320
SUB = 64
NSUB = DPT // SUB
NPAD = NW * DPT
CHUNK = 128
EPAD = E + CHUNK
NEG = -3.0e38


def _dot(a, b):
    return jax.lax.dot_general(a, b, (((1,), (0,)), ((), ())),
                               preferred_element_type=jnp.float32)


def _embed_body(x_ref, we_ref, be_ref, wt_ref, wb_ref, h_ref, a_ref, b_ref):
    h = _dot(x_ref[...], we_ref[...]) + be_ref[...]
    h_ref[...] = h
    a_ref[...] = _dot(h, wt_ref[...])
    b_ref[...] = _dot(h, wb_ref[...])


def _embed_call(x, we, be, wt, wb):
    spec_r = pl.BlockSpec((ROWS, D), lambda i: (i, 0))
    spec_w = pl.BlockSpec((D, D), lambda i: (0, 0))
    spec_b = pl.BlockSpec((1, D), lambda i: (0, 0))
    return pl.pallas_call(
        _embed_body,
        grid=(GRID,),
        in_specs=[spec_r, spec_w, spec_b, spec_w, spec_w],
        out_specs=[spec_r, spec_r, spec_r],
        out_shape=[jax.ShapeDtypeStruct((N, D), jnp.float32)] * 3,
    )(x, we, be.reshape(1, D), wt, wb)


def _post_body(h_ref, b_ref, s_ref, m_ref, hr_ref, nw_ref, bpre_ref,
               wp_ref, bpost_ref, h2_ref, sum_ref, sumsq_ref):
    i = pl.program_id(0)
    h = h_ref[...]
    deg = nw_ref[..., 0:1]
    wsum = nw_ref[..., 1:2]
    awsum = nw_ref[..., 2:3]
    snorm = nw_ref[..., 3:4]
    bb = b_ref[...] + bpre_ref[...]
    agg_sum = s_ref[...] + deg * bb
    agg_mean = agg_sum / jnp.maximum(deg, 1.0)
    agg_max = jnp.where(deg > 0, m_ref[...] + bb, 0.0)
    denom = awsum + 1e-8
    sw = wsum / denom
    agg_dir = jnp.abs(hr_ref[...] / denom + sw * (bb - h))
    h2 = (_dot(h, wp_ref[0]) + _dot(agg_mean, wp_ref[1])
          + _dot(agg_sum, wp_ref[2]) + _dot(agg_max, wp_ref[3])
          + _dot(agg_dir, wp_ref[4]) + bpost_ref[...])
    h2 = h2 * snorm
    h2_ref[...] = h2

    @pl.when(i == 0)
    def _():
        sum_ref[...] = jnp.zeros_like(sum_ref)
        sumsq_ref[...] = jnp.zeros_like(sumsq_ref)

    sum_ref[...] += jnp.sum(h2, axis=0, keepdims=True)
    sumsq_ref[...] += jnp.sum(h2 * h2, axis=0, keepdims=True)


def _post_call(h, b, s, m, hr, nodew, bpre, wp, bpost):
    spec_r = pl.BlockSpec((ROWS, D), lambda i: (i, 0))
    spec_n = pl.BlockSpec((ROWS, 4), lambda i: (i, 0))
    spec_b = pl.BlockSpec((1, D), lambda i: (0, 0))
    spec_wp = pl.BlockSpec((5, D, D), lambda i: (0, 0, 0))
    return pl.pallas_call(
        _post_body,
        grid=(GRID,),
        in_specs=[spec_r, spec_r, spec_r, spec_r, spec_r, spec_n, spec_b,
                  spec_wp, spec_b],
        out_specs=[spec_r, spec_b, spec_b],
        out_shape=[jax.ShapeDtypeStruct((N, D), jnp.float32),
                   jax.ShapeDtypeStruct((1, D), jnp.float32),
                   jax.ShapeDtypeStruct((1, D), jnp.float32)],
    )(h, b, s, m, hr, nodew, bpre.reshape(1, D), wp, bpost.reshape(1, D))


def _layer_core(h, b_, s_, m_, hr_, nw_, bpre, wp_ref, bpost):
    deg = nw_[..., 0:1]
    wsum = nw_[..., 1:2]
    awsum = nw_[..., 2:3]
    snorm = nw_[..., 3:4]
    bb = b_ + bpre
    agg_sum = s_ + deg * bb
    agg_mean = agg_sum / jnp.maximum(deg, 1.0)
    agg_max = jnp.where(deg > 0, m_ + bb, 0.0)
    denom = awsum + 1e-8
    sw = wsum / denom
    agg_dir = jnp.abs(hr_ / denom + sw * (bb - h))
    h2 = (_dot(h, wp_ref[0]) + _dot(agg_mean, wp_ref[1])
          + _dot(agg_sum, wp_ref[2]) + _dot(agg_max, wp_ref[3])
          + _dot(agg_dir, wp_ref[4]) + bpost)
    return h2 * snorm


def _layer_body(h_ref, b_ref, s_ref, m_ref, hr_ref, nw_ref, bpre_ref,
                wp_ref, bpost_ref, gam_ref, bet_ref, wtn_ref, wbn_ref,
                hn_ref, a_ref, bout_ref, h2s_ref, stats_ref):
    p = pl.program_id(0)
    i = pl.program_id(1)

    @pl.when(p == 0)
    def _():
        h2 = _layer_core(h_ref[...], b_ref[...], s_ref[...], m_ref[...],
                         hr_ref[...], nw_ref[...], bpre_ref[...], wp_ref,
                         bpost_ref[...])
        h2s_ref[pl.ds(i * ROWS, ROWS), :] = h2

        @pl.when(i == 0)
        def _():
            stats_ref[...] = jnp.zeros_like(stats_ref)

        stats_ref[0:1, :] += jnp.sum(h2, axis=0, keepdims=True)
        stats_ref[1:2, :] += jnp.sum(h2 * h2, axis=0, keepdims=True)

    @pl.when(p == 1)
    def _():
        mu = stats_ref[0:1, :] * (1.0 / N)
        var = stats_ref[1:2, :] * (1.0 / N) - mu * mu
        h2 = h2s_ref[pl.ds(i * ROWS, ROWS), :]
        hn = (h2 - mu) * jax.lax.rsqrt(var + 1e-5) * gam_ref[...] + bet_ref[...]
        hn = jnp.maximum(hn, 0.0) + h_ref[...]
        hn_ref[...] = hn
        a_ref[...] = _dot(hn, wtn_ref[...])
        bout_ref[...] = _dot(hn, wbn_ref[...])


def _layer_call(h, b, sm, mm, hr, nodew, bpre, wp, bpost, gam, bet, wtn, wbn):
    spec_r = pl.BlockSpec((ROWS, D), lambda p, i: (i, 0))
    spec_n = pl.BlockSpec((ROWS, 4), lambda p, i: (i, 0))
    spec_b = pl.BlockSpec((1, D), lambda p, i: (0, 0))
    spec_w = pl.BlockSpec((D, D), lambda p, i: (0, 0))
    spec_wp = pl.BlockSpec((5, D, D), lambda p, i: (0, 0, 0))
    return pl.pallas_call(
        _layer_body,
        grid=(2, GRID),
        in_specs=[spec_r, spec_r, spec_r, spec_r, spec_r, spec_n, spec_b,
                  spec_wp, spec_b, spec_b, spec_b, spec_w, spec_w],
        out_specs=[spec_r, spec_r, spec_r],
        out_shape=[jax.ShapeDtypeStruct((N, D), jnp.float32)] * 3,
        scratch_shapes=[pltpu.VMEM((N, D), jnp.float32),
                        pltpu.VMEM((8, D), jnp.float32)],
    )(h, b, sm, mm, hr, nodew, bpre.reshape(1, D), wp, bpost.reshape(1, D),
      gam.reshape(1, D), bet.reshape(1, D), wtn, wbn)


def _layer_readout_body(h_ref, b_ref, s_ref, m_ref, hr_ref, nw_ref, bpre_ref,
                        wp_ref, bpost_ref, gam_ref, bet_ref,
                        w0_ref, b0_ref, w1_ref, b1_ref, w2_ref, b2_ref,
                        y_ref, h2s_ref, stats_ref):
    p = pl.program_id(0)
    i = pl.program_id(1)

    @pl.when(p == 0)
    def _():
        h2 = _layer_core(h_ref[...], b_ref[...], s_ref[...], m_ref[...],
                         hr_ref[...], nw_ref[...], bpre_ref[...], wp_ref,
                         bpost_ref[...])
        h2s_ref[pl.ds(i * ROWS, ROWS), :] = h2

        @pl.when(i == 0)
        def _():
            stats_ref[...] = jnp.zeros_like(stats_ref)

        stats_ref[0:1, :] += jnp.sum(h2, axis=0, keepdims=True)
        stats_ref[1:2, :] += jnp.sum(h2 * h2, axis=0, keepdims=True)

    @pl.when(p == 1)
    def _():
        mu = stats_ref[0:1, :] * (1.0 / N)
        var = stats_ref[1:2, :] * (1.0 / N) - mu * mu
        h2 = h2s_ref[pl.ds(i * ROWS, ROWS), :]
        hn = (h2 - mu) * jax.lax.rsqrt(var + 1e-5) * gam_ref[...] + bet_ref[...]
        hn = jnp.maximum(hn, 0.0) + h_ref[...]
        y = jnp.maximum(_dot(hn, w0_ref[...]) + b0_ref[...], 0.0)
        y = jnp.maximum(_dot(y, w1_ref[...]) + b1_ref[...], 0.0)
        y_ref[...] = _dot(y, w2_ref[...]) + b2_ref[...]


def _layer_readout_call(h, b, sm, mm, hr, nodew, bpre, wp, bpost, gam, bet, ro):
    spec_r = pl.BlockSpec((ROWS, D), lambda p, i: (i, 0))
    spec_n = pl.BlockSpec((ROWS, 4), lambda p, i: (i, 0))
    spec_b = pl.BlockSpec((1, D), lambda p, i: (0, 0))
    spec_w = pl.BlockSpec((D, D), lambda p, i: (0, 0))
    spec_wp = pl.BlockSpec((5, D, D), lambda p, i: (0, 0, 0))
    (w0, b0), (w1, b1), (w2, b2) = ro
    w0p = jnp.zeros((D, D), jnp.float32).at[:, :64].set(w0)
    b0p = jnp.zeros((1, D), jnp.float32).at[:, :64].set(b0)
    w1p = jnp.zeros((D, D), jnp.float32).at[:64, :32].set(w1)
    b1p = jnp.zeros((1, D), jnp.float32).at[:, :32].set(b1)
    w2p = jnp.zeros((D, D), jnp.float32).at[:32, :2].set(w2)
    b2p = jnp.zeros((1, D), jnp.float32).at[:, :2].set(b2)
    return pl.pallas_call(
        _layer_readout_body,
        grid=(2, GRID),
        in_specs=[spec_r, spec_r, spec_r, spec_r, spec_r, spec_n, spec_b,
                  spec_wp, spec_b, spec_b, spec_b,
                  spec_w, spec_b, spec_w, spec_b, spec_w, spec_b],
        out_specs=spec_r,
        out_shape=jax.ShapeDtypeStruct((N, D), jnp.float32),
        scratch_shapes=[pltpu.VMEM((N, D), jnp.float32),
                        pltpu.VMEM((8, D), jnp.float32)],
    )(h, b, sm, mm, hr, nodew, bpre.reshape(1, D), wp, bpost.reshape(1, D),
      gam.reshape(1, D), bet.reshape(1, D), w0p, b0p, w1p, b1p, w2p, b2p)


def _bn(h2_ref, hin_ref, sum_ref, sumsq_ref, gam_ref, bet_ref):
    mu = sum_ref[...] * (1.0 / N)
    var = sumsq_ref[...] * (1.0 / N) - mu * mu
    hn = (h2_ref[...] - mu) * jax.lax.rsqrt(var + 1e-5) * gam_ref[...] + bet_ref[...]
    return jnp.maximum(hn, 0.0) + hin_ref[...]


def _bn_ab_body(h2_ref, hin_ref, sum_ref, sumsq_ref, gam_ref, bet_ref,
                wt_ref, wb_ref, hn_ref, a_ref, b_ref):
    hn = _bn(h2_ref, hin_ref, sum_ref, sumsq_ref, gam_ref, bet_ref)
    hn_ref[...] = hn
    a_ref[...] = _dot(hn, wt_ref[...])
    b_ref[...] = _dot(hn, wb_ref[...])


def _bn_ab_call(h2, hin, sums, sumsq, gam, bet, wt, wb):
    spec_r = pl.BlockSpec((ROWS, D), lambda i: (i, 0))
    spec_b = pl.BlockSpec((1, D), lambda i: (0, 0))
    spec_w = pl.BlockSpec((D, D), lambda i: (0, 0))
    return pl.pallas_call(
        _bn_ab_body,
        grid=(GRID,),
        in_specs=[spec_r, spec_r, spec_b, spec_b, spec_b, spec_b, spec_w,
                  spec_w],
        out_specs=[spec_r, spec_r, spec_r],
        out_shape=[jax.ShapeDtypeStruct((N, D), jnp.float32)] * 3,
    )(h2, hin, sums, sumsq, gam.reshape(1, D), bet.reshape(1, D), wt, wb)


def _bn_readout_body(h2_ref, hin_ref, sum_ref, sumsq_ref, gam_ref, bet_ref,
                     w0_ref, b0_ref, w1_ref, b1_ref, w2_ref, b2_ref, y_ref):
    hn = _bn(h2_ref, hin_ref, sum_ref, sumsq_ref, gam_ref, bet_ref)
    y = jnp.maximum(_dot(hn, w0_ref[...]) + b0_ref[...], 0.0)
    y = jnp.maximum(_dot(y, w1_ref[...]) + b1_ref[...], 0.0)
    y_ref[...] = _dot(y, w2_ref[...]) + b2_ref[...]


def _bn_readout_call(h2, hin, sums, sumsq, gam, bet, ro):
    spec_r = pl.BlockSpec((ROWS, D), lambda i: (i, 0))
    spec_b = pl.BlockSpec((1, D), lambda i: (0, 0))
    spec_w = pl.BlockSpec((D, D), lambda i: (0, 0))
    (w0, b0), (w1, b1), (w2, b2) = ro
    w0p = jnp.zeros((D, D), jnp.float32).at[:, :64].set(w0)
    b0p = jnp.zeros((1, D), jnp.float32).at[:, :64].set(b0)
    w1p = jnp.zeros((D, D), jnp.float32).at[:64, :32].set(w1)
    b1p = jnp.zeros((1, D), jnp.float32).at[:, :32].set(b1)
    w2p = jnp.zeros((D, D), jnp.float32).at[:32, :2].set(w2)
    b2p = jnp.zeros((1, D), jnp.float32).at[:, :2].set(b2)
    return pl.pallas_call(
        _bn_readout_body,
        grid=(GRID,),
        in_specs=[spec_r, spec_r, spec_b, spec_b, spec_b, spec_b,
                  spec_w, spec_b, spec_w, spec_b, spec_w, spec_b],
        out_specs=spec_r,
        out_shape=jax.ShapeDtypeStruct((N, D), jnp.float32),
    )(h2, hin, sums, sumsq, gam.reshape(1, D), bet.reshape(1, D),
      w0p, b0p, w1p, b1p, w2p, b2p)


_SC_MESH = None


def _sc_mesh():
    global _SC_MESH
    if _SC_MESH is None:
        _SC_MESH = plsc.VectorSubcoreMesh(core_axis_name="c", subcore_axis_name="s",
                                          num_cores=NC, num_subcores=NS)
    return _SC_MESH


def _sc_body(a_hbm, src_hbm, dst_hbm, w_hbm, st_hbm,
             s_hbm, m_hbm, hr_hbm, nsc_hbm,
             st_s, dsm, wsm, st_v, src_v, dst_v, w_v, rows_v,
             stS, stM, stH, stN,
             sem_i1, sem_i2, sem_i3, sem_g):
    wid = lax.axis_index("s") * NC + lax.axis_index("c")
    d_base = wid * DPT
    pltpu.sync_copy(st_hbm.at[wid], st_v)
    stvec = st_v[...]
    for k in range(NSUB + 1):
        st_s[k] = stvec[k]

    zero16 = jnp.zeros((16,), jnp.float32)
    one16 = jnp.ones((16,), jnp.float32)
    neg16 = jnp.full((16,), NEG, jnp.float32)

    def zero_accs():
        return ([zero16] * 8, [neg16] * 8, [zero16] * 8, [zero16] * 3)

    def sub_body(j, _):
        sub_base = d_base + j * SUB

        def zrow(r, c_):
            base = r * D
            for c in range(8):
                stS[pl.ds(base + c * 16, 16)] = zero16
                stH[pl.ds(base + c * 16, 16)] = zero16
            for q in range(3):
                stN[pl.ds(q * SUB * 16 + r * 16, 16)] = zero16
            return c_

        lax.fori_loop(0, SUB, zrow, 0)

        s0 = st_s[j]
        s1 = st_s[j + 1]
        a0 = (s0 // 8) * 8
        nch = (s1 - a0 + CHUNK - 1) // CHUNK

        def flush(cur_ld, accS, accM, accH, accN):
            @pl.when(jnp.logical_and(cur_ld >= 0, cur_ld < SUB))
            def _():
                base = cur_ld * D
                for c in range(8):
                    stS[pl.ds(base + c * 16, 16)] = accS[c]
                    stM[pl.ds(base + c * 16, 16)] = accM[c]
                    stH[pl.ds(base + c * 16, 16)] = accH[c]
                for q in range(3):
                    stN[pl.ds(q * SUB * 16 + cur_ld * 16, 16)] = accN[q]

        def issue_idx(k):
            b = lax.rem(k, 3)
            cb = a0 + jnp.minimum(k, jnp.maximum(nch, 1) - 1) * CHUNK
            dsl = pl.ds(cb, CHUNK)
            pltpu.async_copy(src_hbm.at[dsl], src_v.at[b], sem_i1)
            pltpu.async_copy(dst_hbm.at[dsl], dst_v.at[b], sem_i2)
            pltpu.async_copy(w_hbm.at[dsl], w_v.at[b], sem_i3)

        def wait_idx():
            dsl = pl.ds(0, CHUNK)
            pltpu.make_async_copy(src_hbm.at[dsl], src_v.at[0], sem_i1).wait()
            pltpu.make_async_copy(dst_hbm.at[dsl], dst_v.at[0], sem_i2).wait()
            pltpu.make_async_copy(w_hbm.at[dsl], w_v.at[0], sem_i3).wait()

        def issue_gather(k):
            b = lax.rem(k, 3)
            rb = lax.rem(k, 2)
            pltpu.async_copy(a_hbm.at[src_v.at[b]], rows_v.at[rb], sem_g)

        def wait_gather():
            pltpu.make_async_copy(a_hbm.at[src_v.at[0]], rows_v.at[0],
                                  sem_g).wait()

        def chunk_body(k, carry):
            b = lax.rem(k, 3)
            rb = lax.rem(k, 2)
            wait_gather()
            wait_idx()
            issue_gather(k + 1)
            issue_idx(k + 2)

            def extr(g, c_):
                dst16 = dst_v[b, pl.ds(g * 16, 16)]
                for lane in range(16):
                    dsm[g * 16 + lane] = dst16[lane]
                return c_

            lax.fori_loop(0, CHUNK // 16, extr, 0)

            def edge(i, ec):
                cur_ld, accS, accM, accH, accN = ec
                ld = dsm[i] - sub_base
                new_seg = ld != cur_ld

                @pl.when(new_seg)
                def _():
                    flush(cur_ld, accS, accM, accH, accN)

                g16 = (i // 16) * 16
                w16 = w_v[b, pl.ds(g16, 16)]
                bw = jnp.take_along_axis(
                    w16, jnp.full((16,), i - g16, jnp.int32), axis=0)
                accN = [jnp.where(new_seg, zero16, accN[0]) + one16,
                        jnp.where(new_seg, zero16, accN[1]) + bw,
                        jnp.where(new_seg, zero16, accN[2])
                        + jnp.maximum(bw, -bw)]
                accS = list(accS)
                accM = list(accM)
                accH = list(accH)
                for c in range(8):
                    a = rows_v[rb, i, pl.ds(c * 16, 16)]
                    accS[c] = jnp.where(new_seg, zero16, accS[c]) + a
                    accM[c] = jnp.maximum(jnp.where(new_seg, neg16, accM[c]), a)
                    accH[c] = jnp.where(new_seg, zero16, accH[c]) + bw * a
                return (ld, accS, accM, accH, accN)

            return lax.fori_loop(0, CHUNK, edge, carry)

        carry0 = (jnp.int32(-1),) + zero_accs()
        issue_idx(0)
        wait_idx()
        issue_gather(0)
        issue_idx(1)
        cur_ld, accS, accM, accH, accN = lax.fori_loop(0, nch, chunk_body, carry0)
        wait_gather()
        wait_idx()
        flush(cur_ld, accS, accM, accH, accN)

        pltpu.sync_copy(stS, s_hbm.at[pl.ds(sub_base * D, SUB * D)])
        pltpu.sync_copy(stM, m_hbm.at[pl.ds(sub_base * D, SUB * D)])
        pltpu.sync_copy(stH, hr_hbm.at[pl.ds(sub_base * D, SUB * D)])
        for q in range(3):
            pltpu.sync_copy(
                stN.at[pl.ds(q * SUB * 16, SUB * 16)],
                nsc_hbm.at[pl.ds(q * NPAD * 16 + sub_base * 16, SUB * 16)])
        return _

    lax.fori_loop(0, NSUB, sub_body, 0)


def _sc_segment_call(a, srcp, dstp, wpad, st2d):
    f = pl.kernel(
        _sc_body,
        out_type=[jax.ShapeDtypeStruct((NPAD * D,), jnp.float32),
                  jax.ShapeDtypeStruct((NPAD * D,), jnp.float32),
                  jax.ShapeDtypeStruct((NPAD * D,), jnp.float32),
                  jax.ShapeDtypeStruct((3 * NPAD * 16,), jnp.float32)],
        mesh=_sc_mesh(),
        scratch_types=[
            pltpu.SMEM((16,), jnp.int32),
            pltpu.SMEM((CHUNK,), jnp.int32),
            pltpu.SMEM((CHUNK,), jnp.float32),
            pltpu.VMEM((16,), jnp.int32),
            pltpu.VMEM((3, CHUNK), jnp.int32),
            pltpu.VMEM((3, CHUNK), jnp.int32),
            pltpu.VMEM((3, CHUNK), jnp.float32),
            pltpu.VMEM((2, CHUNK, D), jnp.float32),
            pltpu.VMEM((SUB * D,), jnp.float32),
            pltpu.VMEM((SUB * D,), jnp.float32),
            pltpu.VMEM((SUB * D,), jnp.float32),
            pltpu.VMEM((3 * SUB * 16,), jnp.float32),
            pltpu.SemaphoreType.DMA,
            pltpu.SemaphoreType.DMA,
            pltpu.SemaphoreType.DMA,
            pltpu.SemaphoreType.DMA,
        ],
    )
    s, m, hr, nsc = f(a, srcp, dstp, wpad, st2d)
    return (s.reshape(NPAD, D), m.reshape(NPAD, D), hr.reshape(NPAD, D),
            nsc.reshape(3, NPAD, 16))


def kernel(x, edge_index, eig, snorm_n, params):
    src = edge_index[0]
    dst = edge_index[1]
    w = eig[:, EIG_IDX]

    key, w_s = jax.lax.sort((dst * 16384 + src, w), num_keys=1,
                            is_stable=False)
    dst_s = key // 16384
    src_s = key - dst_s * 16384
    srcp = jnp.concatenate([src_s, jnp.zeros((CHUNK,), jnp.int32)])
    dstp = jnp.concatenate([dst_s, jnp.full((CHUNK,), NPAD, jnp.int32)])
    wpad = jnp.concatenate([w_s, jnp.zeros((CHUNK,), jnp.float32)])
    bounds = jnp.searchsorted(
        dst_s, (jnp.arange(NW * NSUB + 1) * SUB).astype(jnp.int32)).astype(jnp.int32)
    bidx = jnp.minimum(jnp.arange(NW)[:, None] * NSUB + jnp.arange(16)[None, :],
                       NW * NSUB)
    st2d = bounds[bidx].astype(jnp.int32)

    lp0 = params['layers'][0]
    h, a, b = _embed_call(x, params['W_embed'], params['b_embed'],
                          lp0['W_pre'][:D], lp0['W_pre'][D:])

    n_layers = len(params['layers'])
    nodew = None
    for li, lp in enumerate(params['layers']):
        s, m, hr, nsc = _sc_segment_call(a, srcp, dstp, wpad, st2d)
        s, m, hr = s[:N], m[:N], hr[:N]
        if nodew is None:
            nodew = jnp.stack([nsc[0, :N, 0], nsc[1, :N, 0],
                               nsc[2, :N, 0], snorm_n[:, 0]], axis=1)
        wp = jnp.stack([lp['W_post'][0:D], lp['W_post'][D:2 * D],
                        lp['W_post'][2 * D:3 * D], lp['W_post'][3 * D:4 * D],
                        lp['W_post'][4 * D:5 * D]], axis=0)
        if li + 1 < n_layers:
            lpn = params['layers'][li + 1]
            h, a, b = _layer_call(h, b, s, m, hr, nodew, lp['b_pre'], wp,
                                  lp['b_post'], lp['gamma'], lp['beta'],
                                  lpn['W_pre'][:D], lpn['W_pre'][D:])
        else:
            y = _layer_readout_call(h, b, s, m, hr, nodew, lp['b_pre'], wp,
                                    lp['b_post'], lp['gamma'], lp['beta'],
                                    params['readout'])
    return y[:, :2]


# speedup vs baseline: 1.0007x; 1.0007x over previous
import functools

import jax
import jax.numpy as jnp
from jax import lax
from jax.experimental import pallas as pl
from jax.experimental.pallas import tpu as pltpu
from jax.experimental.pallas import tpu_sc as plsc

N = 10000
E = 320000
D = 128
EIG_IDX = 1
ROWS = 1000
GRID = N // ROWS

NC = 2
NS = 16
NW = NC * NS
DPT = 320
SUB = 64
NSUB = DPT // SUB
NPAD = NW * DPT
CHUNK = 128
EPAD = E + CHUNK
NEG = -3.0e38


def _dot(a, b):
    return jax.lax.dot_general(a, b, (((1,), (0,)), ((), ())),
                               preferred_element_type=jnp.float32)


def _embed_body(x_ref, we_ref, be_ref, wt_ref, wb_ref, h_ref, a_ref, b_ref):
    h = _dot(x_ref[...], we_ref[...]) + be_ref[...]
    h_ref[...] = h
    a_ref[...] = _dot(h, wt_ref[...])
    b_ref[...] = _dot(h, wb_ref[...])


def _embed_call(x, we, be, wt, wb):
    spec_r = pl.BlockSpec((ROWS, D), lambda i: (i, 0))
    spec_w = pl.BlockSpec((D, D), lambda i: (0, 0))
    spec_b = pl.BlockSpec((1, D), lambda i: (0, 0))
    return pl.pallas_call(
        _embed_body,
        grid=(GRID,),
        in_specs=[spec_r, spec_w, spec_b, spec_w, spec_w],
        out_specs=[spec_r, spec_r, spec_r],
        out_shape=[jax.ShapeDtypeStruct((N, D), jnp.float32)] * 3,
    )(x, we, be.reshape(1, D), wt, wb)


def _post_body(h_ref, b_ref, s_ref, m_ref, hr_ref, nw_ref, bpre_ref,
               wp_ref, bpost_ref, h2_ref, sum_ref, sumsq_ref):
    i = pl.program_id(0)
    h = h_ref[...]
    deg = nw_ref[..., 0:1]
    wsum = nw_ref[..., 1:2]
    awsum = nw_ref[..., 2:3]
    snorm = nw_ref[..., 3:4]
    bb = b_ref[...] + bpre_ref[...]
    agg_sum = s_ref[...] + deg * bb
    agg_mean = agg_sum / jnp.maximum(deg, 1.0)
    agg_max = jnp.where(deg > 0, m_ref[...] + bb, 0.0)
    denom = awsum + 1e-8
    sw = wsum / denom
    agg_dir = jnp.abs(hr_ref[...] / denom + sw * (bb - h))
    h2 = (_dot(h, wp_ref[0]) + _dot(agg_mean, wp_ref[1])
          + _dot(agg_sum, wp_ref[2]) + _dot(agg_max, wp_ref[3])
          + _dot(agg_dir, wp_ref[4]) + bpost_ref[...])
    h2 = h2 * snorm
    h2_ref[...] = h2

    @pl.when(i == 0)
    def _():
        sum_ref[...] = jnp.zeros_like(sum_ref)
        sumsq_ref[...] = jnp.zeros_like(sumsq_ref)

    sum_ref[...] += jnp.sum(h2, axis=0, keepdims=True)
    sumsq_ref[...] += jnp.sum(h2 * h2, axis=0, keepdims=True)


def _post_call(h, b, s, m, hr, nodew, bpre, wp, bpost):
    spec_r = pl.BlockSpec((ROWS, D), lambda i: (i, 0))
    spec_n = pl.BlockSpec((ROWS, 4), lambda i: (i, 0))
    spec_b = pl.BlockSpec((1, D), lambda i: (0, 0))
    spec_wp = pl.BlockSpec((5, D, D), lambda i: (0, 0, 0))
    return pl.pallas_call(
        _post_body,
        grid=(GRID,),
        in_specs=[spec_r, spec_r, spec_r, spec_r, spec_r, spec_n, spec_b,
                  spec_wp, spec_b],
        out_specs=[spec_r, spec_b, spec_b],
        out_shape=[jax.ShapeDtypeStruct((N, D), jnp.float32),
                   jax.ShapeDtypeStruct((1, D), jnp.float32),
                   jax.ShapeDtypeStruct((1, D), jnp.float32)],
    )(h, b, s, m, hr, nodew, bpre.reshape(1, D), wp, bpost.reshape(1, D))


def _layer_core(h, b_, s_, m_, hr_, nw_, bpre, wp_ref, bpost):
    deg = nw_[..., 0:1]
    wsum = nw_[..., 1:2]
    awsum = nw_[..., 2:3]
    snorm = nw_[..., 3:4]
    bb = b_ + bpre
    agg_sum = s_ + deg * bb
    agg_mean = agg_sum / jnp.maximum(deg, 1.0)
    agg_max = jnp.where(deg > 0, m_ + bb, 0.0)
    denom = awsum + 1e-8
    sw = wsum / denom
    agg_dir = jnp.abs(hr_ / denom + sw * (bb - h))
    h2 = (_dot(h, wp_ref[0]) + _dot(agg_mean, wp_ref[1])
          + _dot(agg_sum, wp_ref[2]) + _dot(agg_max, wp_ref[3])
          + _dot(agg_dir, wp_ref[4]) + bpost)
    return h2 * snorm


def _layer_body(h_ref, b_ref, s_ref, m_ref, hr_ref, nw_ref, bpre_ref,
                wp_ref, bpost_ref, gam_ref, bet_ref, wtn_ref, wbn_ref,
                hn_ref, a_ref, bout_ref, h2s_ref, stats_ref):
    p = pl.program_id(0)
    i = pl.program_id(1)

    @pl.when(p == 0)
    def _():
        h2 = _layer_core(h_ref[...], b_ref[...], s_ref[...], m_ref[...],
                         hr_ref[...], nw_ref[...], bpre_ref[...], wp_ref,
                         bpost_ref[...])
        h2s_ref[pl.ds(i * ROWS, ROWS), :] = h2

        @pl.when(i == 0)
        def _():
            stats_ref[...] = jnp.zeros_like(stats_ref)

        stats_ref[0:1, :] += jnp.sum(h2, axis=0, keepdims=True)
        stats_ref[1:2, :] += jnp.sum(h2 * h2, axis=0, keepdims=True)

    @pl.when(p == 1)
    def _():
        mu = stats_ref[0:1, :] * (1.0 / N)
        var = stats_ref[1:2, :] * (1.0 / N) - mu * mu
        h2 = h2s_ref[pl.ds(i * ROWS, ROWS), :]
        hn = (h2 - mu) * jax.lax.rsqrt(var + 1e-5) * gam_ref[...] + bet_ref[...]
        hn = jnp.maximum(hn, 0.0) + h_ref[...]
        hn_ref[...] = hn
        a_ref[...] = _dot(hn, wtn_ref[...])
        bout_ref[...] = _dot(hn, wbn_ref[...])


def _layer_call(h, b, sm, mm, hr, nodew, bpre, wp, bpost, gam, bet, wtn, wbn):
    spec_r = pl.BlockSpec((ROWS, D), lambda p, i: (i, 0))
    spec_n = pl.BlockSpec((ROWS, 4), lambda p, i: (i, 0))
    spec_b = pl.BlockSpec((1, D), lambda p, i: (0, 0))
    spec_w = pl.BlockSpec((D, D), lambda p, i: (0, 0))
    spec_wp = pl.BlockSpec((5, D, D), lambda p, i: (0, 0, 0))
    return pl.pallas_call(
        _layer_body,
        grid=(2, GRID),
        in_specs=[spec_r, spec_r, spec_r, spec_r, spec_r, spec_n, spec_b,
                  spec_wp, spec_b, spec_b, spec_b, spec_w, spec_w],
        out_specs=[spec_r, spec_r, spec_r],
        out_shape=[jax.ShapeDtypeStruct((N, D), jnp.float32)] * 3,
        scratch_shapes=[pltpu.VMEM((N, D), jnp.float32),
                        pltpu.VMEM((8, D), jnp.float32)],
    )(h, b, sm, mm, hr, nodew, bpre.reshape(1, D), wp, bpost.reshape(1, D),
      gam.reshape(1, D), bet.reshape(1, D), wtn, wbn)


def _layer_readout_body(h_ref, b_ref, s_ref, m_ref, hr_ref, nw_ref, bpre_ref,
                        wp_ref, bpost_ref, gam_ref, bet_ref,
                        w0_ref, b0_ref, w1_ref, b1_ref, w2_ref, b2_ref,
                        y_ref, h2s_ref, stats_ref):
    p = pl.program_id(0)
    i = pl.program_id(1)

    @pl.when(p == 0)
    def _():
        h2 = _layer_core(h_ref[...], b_ref[...], s_ref[...], m_ref[...],
                         hr_ref[...], nw_ref[...], bpre_ref[...], wp_ref,
                         bpost_ref[...])
        h2s_ref[pl.ds(i * ROWS, ROWS), :] = h2

        @pl.when(i == 0)
        def _():
            stats_ref[...] = jnp.zeros_like(stats_ref)

        stats_ref[0:1, :] += jnp.sum(h2, axis=0, keepdims=True)
        stats_ref[1:2, :] += jnp.sum(h2 * h2, axis=0, keepdims=True)

    @pl.when(p == 1)
    def _():
        mu = stats_ref[0:1, :] * (1.0 / N)
        var = stats_ref[1:2, :] * (1.0 / N) - mu * mu
        h2 = h2s_ref[pl.ds(i * ROWS, ROWS), :]
        hn = (h2 - mu) * jax.lax.rsqrt(var + 1e-5) * gam_ref[...] + bet_ref[...]
        hn = jnp.maximum(hn, 0.0) + h_ref[...]
        y = jnp.maximum(_dot(hn, w0_ref[...]) + b0_ref[...], 0.0)
        y = jnp.maximum(_dot(y, w1_ref[...]) + b1_ref[...], 0.0)
        y_ref[...] = _dot(y, w2_ref[...]) + b2_ref[...]


def _layer_readout_call(h, b, sm, mm, hr, nodew, bpre, wp, bpost, gam, bet, ro):
    spec_r = pl.BlockSpec((ROWS, D), lambda p, i: (i, 0))
    spec_n = pl.BlockSpec((ROWS, 4), lambda p, i: (i, 0))
    spec_b = pl.BlockSpec((1, D), lambda p, i: (0, 0))
    spec_w = pl.BlockSpec((D, D), lambda p, i: (0, 0))
    spec_wp = pl.BlockSpec((5, D, D), lambda p, i: (0, 0, 0))
    (w0, b0), (w1, b1), (w2, b2) = ro
    w0p = jnp.zeros((D, D), jnp.float32).at[:, :64].set(w0)
    b0p = jnp.zeros((1, D), jnp.float32).at[:, :64].set(b0)
    w1p = jnp.zeros((D, D), jnp.float32).at[:64, :32].set(w1)
    b1p = jnp.zeros((1, D), jnp.float32).at[:, :32].set(b1)
    w2p = jnp.zeros((D, D), jnp.float32).at[:32, :2].set(w2)
    b2p = jnp.zeros((1, D), jnp.float32).at[:, :2].set(b2)
    return pl.pallas_call(
        _layer_readout_body,
        grid=(2, GRID),
        in_specs=[spec_r, spec_r, spec_r, spec_r, spec_r, spec_n, spec_b,
                  spec_wp, spec_b, spec_b, spec_b,
                  spec_w, spec_b, spec_w, spec_b, spec_w, spec_b],
        out_specs=spec_r,
        out_shape=jax.ShapeDtypeStruct((N, D), jnp.float32),
        scratch_shapes=[pltpu.VMEM((N, D), jnp.float32),
                        pltpu.VMEM((8, D), jnp.float32)],
    )(h, b, sm, mm, hr, nodew, bpre.reshape(1, D), wp, bpost.reshape(1, D),
      gam.reshape(1, D), bet.reshape(1, D), w0p, b0p, w1p, b1p, w2p, b2p)


def _bn(h2_ref, hin_ref, sum_ref, sumsq_ref, gam_ref, bet_ref):
    mu = sum_ref[...] * (1.0 / N)
    var = sumsq_ref[...] * (1.0 / N) - mu * mu
    hn = (h2_ref[...] - mu) * jax.lax.rsqrt(var + 1e-5) * gam_ref[...] + bet_ref[...]
    return jnp.maximum(hn, 0.0) + hin_ref[...]


def _bn_ab_body(h2_ref, hin_ref, sum_ref, sumsq_ref, gam_ref, bet_ref,
                wt_ref, wb_ref, hn_ref, a_ref, b_ref):
    hn = _bn(h2_ref, hin_ref, sum_ref, sumsq_ref, gam_ref, bet_ref)
    hn_ref[...] = hn
    a_ref[...] = _dot(hn, wt_ref[...])
    b_ref[...] = _dot(hn, wb_ref[...])


def _bn_ab_call(h2, hin, sums, sumsq, gam, bet, wt, wb):
    spec_r = pl.BlockSpec((ROWS, D), lambda i: (i, 0))
    spec_b = pl.BlockSpec((1, D), lambda i: (0, 0))
    spec_w = pl.BlockSpec((D, D), lambda i: (0, 0))
    return pl.pallas_call(
        _bn_ab_body,
        grid=(GRID,),
        in_specs=[spec_r, spec_r, spec_b, spec_b, spec_b, spec_b, spec_w,
                  spec_w],
        out_specs=[spec_r, spec_r, spec_r],
        out_shape=[jax.ShapeDtypeStruct((N, D), jnp.float32)] * 3,
    )(h2, hin, sums, sumsq, gam.reshape(1, D), bet.reshape(1, D), wt, wb)


def _bn_readout_body(h2_ref, hin_ref, sum_ref, sumsq_ref, gam_ref, bet_ref,
                     w0_ref, b0_ref, w1_ref, b1_ref, w2_ref, b2_ref, y_ref):
    hn = _bn(h2_ref, hin_ref, sum_ref, sumsq_ref, gam_ref, bet_ref)
    y = jnp.maximum(_dot(hn, w0_ref[...]) + b0_ref[...], 0.0)
    y = jnp.maximum(_dot(y, w1_ref[...]) + b1_ref[...], 0.0)
    y_ref[...] = _dot(y, w2_ref[...]) + b2_ref[...]


def _bn_readout_call(h2, hin, sums, sumsq, gam, bet, ro):
    spec_r = pl.BlockSpec((ROWS, D), lambda i: (i, 0))
    spec_b = pl.BlockSpec((1, D), lambda i: (0, 0))
    spec_w = pl.BlockSpec((D, D), lambda i: (0, 0))
    (w0, b0), (w1, b1), (w2, b2) = ro
    w0p = jnp.zeros((D, D), jnp.float32).at[:, :64].set(w0)
    b0p = jnp.zeros((1, D), jnp.float32).at[:, :64].set(b0)
    w1p = jnp.zeros((D, D), jnp.float32).at[:64, :32].set(w1)
    b1p = jnp.zeros((1, D), jnp.float32).at[:, :32].set(b1)
    w2p = jnp.zeros((D, D), jnp.float32).at[:32, :2].set(w2)
    b2p = jnp.zeros((1, D), jnp.float32).at[:, :2].set(b2)
    return pl.pallas_call(
        _bn_readout_body,
        grid=(GRID,),
        in_specs=[spec_r, spec_r, spec_b, spec_b, spec_b, spec_b,
                  spec_w, spec_b, spec_w, spec_b, spec_w, spec_b],
        out_specs=spec_r,
        out_shape=jax.ShapeDtypeStruct((N, D), jnp.float32),
    )(h2, hin, sums, sumsq, gam.reshape(1, D), bet.reshape(1, D),
      w0p, b0p, w1p, b1p, w2p, b2p)


_SC_MESH = None


def _sc_mesh():
    global _SC_MESH
    if _SC_MESH is None:
        _SC_MESH = plsc.VectorSubcoreMesh(core_axis_name="c", subcore_axis_name="s",
                                          num_cores=NC, num_subcores=NS)
    return _SC_MESH


def _sc_body(a_hbm, src_hbm, dst_hbm, w_hbm, st_hbm,
             s_hbm, m_hbm, hr_hbm, nsc_hbm,
             st_s, dsm, wsm, st_v, src_v, dst_v, w_v, rows_v,
             stS, stM, stH, stN,
             sem_i1, sem_i2, sem_i3, sem_g):
    wid = lax.axis_index("s") * NC + lax.axis_index("c")
    d_base = wid * DPT
    pltpu.sync_copy(st_hbm.at[wid], st_v)
    stvec = st_v[...]
    for k in range(NSUB + 1):
        st_s[k] = stvec[k]

    zero16 = jnp.zeros((16,), jnp.float32)
    one16 = jnp.ones((16,), jnp.float32)
    neg16 = jnp.full((16,), NEG, jnp.float32)

    def zero_accs():
        return ([zero16] * 8, [neg16] * 8, [zero16] * 8, [zero16] * 3)

    def sub_body(j, _):
        sub_base = d_base + j * SUB

        def zrow(r, c_):
            base = r * D
            for c in range(8):
                stS[pl.ds(base + c * 16, 16)] = zero16
                stH[pl.ds(base + c * 16, 16)] = zero16
            for q in range(3):
                stN[pl.ds(q * SUB * 16 + r * 16, 16)] = zero16
            return c_

        lax.fori_loop(0, SUB, zrow, 0)

        s0 = st_s[j]
        s1 = st_s[j + 1]
        a0 = (s0 // 8) * 8
        nch = (s1 - a0 + CHUNK - 1) // CHUNK

        def flush(cur_ld, accS, accM, accH, accN):
            @pl.when(jnp.logical_and(cur_ld >= 0, cur_ld < SUB))
            def _():
                base = cur_ld * D
                for c in range(8):
                    stS[pl.ds(base + c * 16, 16)] = accS[c]
                    stM[pl.ds(base + c * 16, 16)] = accM[c]
                    stH[pl.ds(base + c * 16, 16)] = accH[c]
                for q in range(3):
                    stN[pl.ds(q * SUB * 16 + cur_ld * 16, 16)] = accN[q]

        def issue_idx(k):
            b = lax.rem(k, 3)
            cb = a0 + jnp.minimum(k, jnp.maximum(nch, 1) - 1) * CHUNK
            dsl = pl.ds(cb, CHUNK)
            pltpu.async_copy(src_hbm.at[dsl], src_v.at[b], sem_i1)
            pltpu.async_copy(dst_hbm.at[dsl], dst_v.at[b], sem_i2)
            pltpu.async_copy(w_hbm.at[dsl], w_v.at[b], sem_i3)

        def wait_idx():
            dsl = pl.ds(0, CHUNK)
            pltpu.make_async_copy(src_hbm.at[dsl], src_v.at[0], sem_i1).wait()
            pltpu.make_async_copy(dst_hbm.at[dsl], dst_v.at[0], sem_i2).wait()
            pltpu.make_async_copy(w_hbm.at[dsl], w_v.at[0], sem_i3).wait()

        def issue_gather(k):
            b = lax.rem(k, 3)
            rb = lax.rem(k, 2)
            pltpu.async_copy(a_hbm.at[src_v.at[b]], rows_v.at[rb], sem_g)

        def wait_gather():
            pltpu.make_async_copy(a_hbm.at[src_v.at[0]], rows_v.at[0],
                                  sem_g).wait()

        def chunk_body(k, carry):
            b = lax.rem(k, 3)
            rb = lax.rem(k, 2)
            wait_gather()
            wait_idx()
            issue_gather(k + 1)
            issue_idx(k + 2)

            def extr(g, c_):
                dst16 = dst_v[b, pl.ds(g * 16, 16)]
                for lane in range(16):
                    dsm[g * 16 + lane] = dst16[lane]
                return c_

            lax.fori_loop(0, CHUNK // 16, extr, 0)

            def edge(i, ec):
                cur_ld, accS, accM, accH, accN = ec
                ld = dsm[i] - sub_base
                new_seg = ld != cur_ld

                def do_flush(accs, _cur=cur_ld):
                    flush(_cur, *accs)
                    return zero_accs()

                accS, accM, accH, accN = lax.cond(
                    new_seg, do_flush, lambda accs: accs,
                    (accS, accM, accH, accN))

                g16 = (i // 16) * 16
                w16 = w_v[b, pl.ds(g16, 16)]
                bw = jnp.take_along_axis(
                    w16, jnp.full((16,), i - g16, jnp.int32), axis=0)
                accN = [accN[0] + one16, accN[1] + bw,
                        accN[2] + jnp.maximum(bw, -bw)]
                accS = list(accS)
                accM = list(accM)
                accH = list(accH)
                for c in range(8):
                    a = rows_v[rb, i, pl.ds(c * 16, 16)]
                    accS[c] = accS[c] + a
                    accM[c] = jnp.maximum(accM[c], a)
                    accH[c] = accH[c] + bw * a
                return (ld, accS, accM, accH, accN)

            return lax.fori_loop(0, CHUNK, edge, carry)

        carry0 = (jnp.int32(-1),) + zero_accs()
        issue_idx(0)
        wait_idx()
        issue_gather(0)
        issue_idx(1)
        cur_ld, accS, accM, accH, accN = lax.fori_loop(0, nch, chunk_body, carry0)
        wait_gather()
        wait_idx()
        flush(cur_ld, accS, accM, accH, accN)

        pltpu.sync_copy(stS, s_hbm.at[pl.ds(sub_base * D, SUB * D)])
        pltpu.sync_copy(stM, m_hbm.at[pl.ds(sub_base * D, SUB * D)])
        pltpu.sync_copy(stH, hr_hbm.at[pl.ds(sub_base * D, SUB * D)])
        for q in range(3):
            pltpu.sync_copy(
                stN.at[pl.ds(q * SUB * 16, SUB * 16)],
                nsc_hbm.at[pl.ds(q * NPAD * 16 + sub_base * 16, SUB * 16)])
        return _

    lax.fori_loop(0, NSUB, sub_body, 0)


def _sc_segment_call(a, srcp, dstp, wpad, st2d):
    f = pl.kernel(
        _sc_body,
        out_type=[jax.ShapeDtypeStruct((NPAD * D,), jnp.float32),
                  jax.ShapeDtypeStruct((NPAD * D,), jnp.float32),
                  jax.ShapeDtypeStruct((NPAD * D,), jnp.float32),
                  jax.ShapeDtypeStruct((3 * NPAD * 16,), jnp.float32)],
        mesh=_sc_mesh(),
        scratch_types=[
            pltpu.SMEM((16,), jnp.int32),
            pltpu.SMEM((CHUNK,), jnp.int32),
            pltpu.SMEM((CHUNK,), jnp.float32),
            pltpu.VMEM((16,), jnp.int32),
            pltpu.VMEM((3, CHUNK), jnp.int32),
            pltpu.VMEM((3, CHUNK), jnp.int32),
            pltpu.VMEM((3, CHUNK), jnp.float32),
            pltpu.VMEM((2, CHUNK, D), jnp.float32),
            pltpu.VMEM((SUB * D,), jnp.float32),
            pltpu.VMEM((SUB * D,), jnp.float32),
            pltpu.VMEM((SUB * D,), jnp.float32),
            pltpu.VMEM((3 * SUB * 16,), jnp.float32),
            pltpu.SemaphoreType.DMA,
            pltpu.SemaphoreType.DMA,
            pltpu.SemaphoreType.DMA,
            pltpu.SemaphoreType.DMA,
        ],
    )
    s, m, hr, nsc = f(a, srcp, dstp, wpad, st2d)
    return (s.reshape(NPAD, D), m.reshape(NPAD, D), hr.reshape(NPAD, D),
            nsc.reshape(3, NPAD, 16))


def kernel(x, edge_index, eig, snorm_n, params):
    src = edge_index[0]
    dst = edge_index[1]
    w = eig[:, EIG_IDX]

    key, w_s = jax.lax.sort((dst * 16384 + src, w), num_keys=1,
                            is_stable=False)
    dst_s = key // 16384
    src_s = key - dst_s * 16384
    srcp = jnp.concatenate([src_s, jnp.zeros((CHUNK,), jnp.int32)])
    dstp = jnp.concatenate([dst_s, jnp.full((CHUNK,), NPAD, jnp.int32)])
    wpad = jnp.concatenate([w_s, jnp.zeros((CHUNK,), jnp.float32)])
    bounds = jnp.searchsorted(
        dst_s, (jnp.arange(NW * NSUB + 1) * SUB).astype(jnp.int32)).astype(jnp.int32)
    bidx = jnp.minimum(jnp.arange(NW)[:, None] * NSUB + jnp.arange(16)[None, :],
                       NW * NSUB)
    st2d = bounds[bidx].astype(jnp.int32)

    lp0 = params['layers'][0]
    h, a, b = _embed_call(x, params['W_embed'], params['b_embed'],
                          lp0['W_pre'][:D], lp0['W_pre'][D:])

    n_layers = len(params['layers'])
    nodew = None
    for li, lp in enumerate(params['layers']):
        s, m, hr, nsc = _sc_segment_call(a, srcp, dstp, wpad, st2d)
        s, m, hr = s[:N], m[:N], hr[:N]
        if nodew is None:
            nodew = jnp.stack([nsc[0, :N, 0], nsc[1, :N, 0],
                               nsc[2, :N, 0], snorm_n[:, 0]], axis=1)
        wp = jnp.stack([lp['W_post'][0:D], lp['W_post'][D:2 * D],
                        lp['W_post'][2 * D:3 * D], lp['W_post'][3 * D:4 * D],
                        lp['W_post'][4 * D:5 * D]], axis=0)
        if li + 1 < n_layers:
            lpn = params['layers'][li + 1]
            h, a, b = _layer_call(h, b, s, m, hr, nodew, lp['b_pre'], wp,
                                  lp['b_post'], lp['gamma'], lp['beta'],
                                  lpn['W_pre'][:D], lpn['W_pre'][D:])
        else:
            y = _layer_readout_call(h, b, s, m, hr, nodew, lp['b_pre'], wp,
                                    lp['b_post'], lp['gamma'], lp['beta'],
                                    params['readout'])
    return y[:, :2]


# speedup vs baseline: 1.0146x; 1.0139x over previous
import jax
import jax.numpy as jnp
from jax import lax
from jax.experimental import pallas as pl
from jax.experimental.pallas import tpu as pltpu
from jax.experimental.pallas import tpu_sc as plsc

N = 10000
E = 320000
D = 128
EIG_IDX = 1
ROWS = 1000
GRID = N // ROWS

NC = 2
NS = 16
NW = NC * NS
DPT = 320
SUB = 64
NSUB = DPT // SUB
NPAD = NW * DPT
CHUNK = 128
EPAD = E + CHUNK
NEG = -3.0e38


def _dot(a, b):
    return jax.lax.dot_general(a, b, (((1,), (0,)), ((), ())),
                               preferred_element_type=jnp.float32)


def _embed_body(x_ref, we_ref, be_ref, wt_ref, wb_ref, h_ref, a_ref, b_ref):
    h = _dot(x_ref[...], we_ref[...]) + be_ref[...]
    h_ref[...] = h
    a_ref[...] = _dot(h, wt_ref[...])
    b_ref[...] = _dot(h, wb_ref[...])


def _embed_call(x, we, be, wt, wb):
    spec_r = pl.BlockSpec((ROWS, D), lambda i: (i, 0))
    spec_w = pl.BlockSpec((D, D), lambda i: (0, 0))
    spec_b = pl.BlockSpec((1, D), lambda i: (0, 0))
    return pl.pallas_call(
        _embed_body,
        grid=(GRID,),
        in_specs=[spec_r, spec_w, spec_b, spec_w, spec_w],
        out_specs=[spec_r, spec_r, spec_r],
        out_shape=[jax.ShapeDtypeStruct((N, D), jnp.float32)] * 3,
    )(x, we, be.reshape(1, D), wt, wb)


def _layer_core(h, b_, s_, m_, hr_, nw_, bpre, wp_ref, bpost):
    deg = nw_[..., 0:1]
    wsum = nw_[..., 1:2]
    awsum = nw_[..., 2:3]
    snorm = nw_[..., 3:4]
    bb = b_ + bpre
    agg_sum = s_ + deg * bb
    agg_mean = agg_sum / jnp.maximum(deg, 1.0)
    agg_max = jnp.where(deg > 0, m_ + bb, 0.0)
    denom = awsum + 1e-8
    sw = wsum / denom
    agg_dir = jnp.abs(hr_ / denom + sw * (bb - h))
    h2 = (_dot(h, wp_ref[0]) + _dot(agg_mean, wp_ref[1])
          + _dot(agg_sum, wp_ref[2]) + _dot(agg_max, wp_ref[3])
          + _dot(agg_dir, wp_ref[4]) + bpost)
    return h2 * snorm


def _layer_body(h_ref, b_ref, s_ref, m_ref, hr_ref, nw_ref, bpre_ref,
                wp_ref, bpost_ref, gam_ref, bet_ref, wtn_ref, wbn_ref,
                hn_ref, a_ref, bout_ref, h2s_ref, stats_ref):
    p = pl.program_id(0)
    i = pl.program_id(1)

    @pl.when(p == 0)
    def _():
        h2 = _layer_core(h_ref[...], b_ref[...], s_ref[...], m_ref[...],
                         hr_ref[...], nw_ref[...], bpre_ref[...], wp_ref,
                         bpost_ref[...])
        h2s_ref[pl.ds(i * ROWS, ROWS), :] = h2

        @pl.when(i == 0)
        def _():
            stats_ref[...] = jnp.zeros_like(stats_ref)

        stats_ref[0:1, :] += jnp.sum(h2, axis=0, keepdims=True)
        stats_ref[1:2, :] += jnp.sum(h2 * h2, axis=0, keepdims=True)

    @pl.when(p == 1)
    def _():
        mu = stats_ref[0:1, :] * (1.0 / N)
        var = stats_ref[1:2, :] * (1.0 / N) - mu * mu
        h2 = h2s_ref[pl.ds(i * ROWS, ROWS), :]
        hn = (h2 - mu) * jax.lax.rsqrt(var + 1e-5) * gam_ref[...] + bet_ref[...]
        hn = jnp.maximum(hn, 0.0) + h_ref[...]
        hn_ref[...] = hn
        a_ref[...] = _dot(hn, wtn_ref[...])
        bout_ref[...] = _dot(hn, wbn_ref[...])


def _layer_call(h, b, sm, mm, hr, nodew, bpre, wp, bpost, gam, bet, wtn, wbn):
    spec_r = pl.BlockSpec((ROWS, D), lambda p, i: (i, 0))
    spec_n = pl.BlockSpec((ROWS, 4), lambda p, i: (i, 0))
    spec_b = pl.BlockSpec((1, D), lambda p, i: (0, 0))
    spec_w = pl.BlockSpec((D, D), lambda p, i: (0, 0))
    spec_wp = pl.BlockSpec((5, D, D), lambda p, i: (0, 0, 0))
    return pl.pallas_call(
        _layer_body,
        grid=(2, GRID),
        in_specs=[spec_r, spec_r, spec_r, spec_r, spec_r, spec_n, spec_b,
                  spec_wp, spec_b, spec_b, spec_b, spec_w, spec_w],
        out_specs=[spec_r, spec_r, spec_r],
        out_shape=[jax.ShapeDtypeStruct((N, D), jnp.float32)] * 3,
        scratch_shapes=[pltpu.VMEM((N, D), jnp.float32),
                        pltpu.VMEM((8, D), jnp.float32)],
    )(h, b, sm, mm, hr, nodew, bpre.reshape(1, D), wp, bpost.reshape(1, D),
      gam.reshape(1, D), bet.reshape(1, D), wtn, wbn)


def _layer_readout_body(h_ref, b_ref, s_ref, m_ref, hr_ref, nw_ref, bpre_ref,
                        wp_ref, bpost_ref, gam_ref, bet_ref,
                        w0_ref, b0_ref, w1_ref, b1_ref, w2_ref, b2_ref,
                        y_ref, h2s_ref, stats_ref):
    p = pl.program_id(0)
    i = pl.program_id(1)

    @pl.when(p == 0)
    def _():
        h2 = _layer_core(h_ref[...], b_ref[...], s_ref[...], m_ref[...],
                         hr_ref[...], nw_ref[...], bpre_ref[...], wp_ref,
                         bpost_ref[...])
        h2s_ref[pl.ds(i * ROWS, ROWS), :] = h2

        @pl.when(i == 0)
        def _():
            stats_ref[...] = jnp.zeros_like(stats_ref)

        stats_ref[0:1, :] += jnp.sum(h2, axis=0, keepdims=True)
        stats_ref[1:2, :] += jnp.sum(h2 * h2, axis=0, keepdims=True)

    @pl.when(p == 1)
    def _():
        mu = stats_ref[0:1, :] * (1.0 / N)
        var = stats_ref[1:2, :] * (1.0 / N) - mu * mu
        h2 = h2s_ref[pl.ds(i * ROWS, ROWS), :]
        hn = (h2 - mu) * jax.lax.rsqrt(var + 1e-5) * gam_ref[...] + bet_ref[...]
        hn = jnp.maximum(hn, 0.0) + h_ref[...]
        y = jnp.maximum(_dot(hn, w0_ref[...]) + b0_ref[...], 0.0)
        y = jnp.maximum(_dot(y, w1_ref[...]) + b1_ref[...], 0.0)
        y_ref[...] = _dot(y, w2_ref[...]) + b2_ref[...]


def _layer_readout_call(h, b, sm, mm, hr, nodew, bpre, wp, bpost, gam, bet, ro):
    spec_r = pl.BlockSpec((ROWS, D), lambda p, i: (i, 0))
    spec_n = pl.BlockSpec((ROWS, 4), lambda p, i: (i, 0))
    spec_b = pl.BlockSpec((1, D), lambda p, i: (0, 0))
    spec_w = pl.BlockSpec((D, D), lambda p, i: (0, 0))
    spec_wp = pl.BlockSpec((5, D, D), lambda p, i: (0, 0, 0))
    (w0, b0), (w1, b1), (w2, b2) = ro
    w0p = jnp.zeros((D, D), jnp.float32).at[:, :64].set(w0)
    b0p = jnp.zeros((1, D), jnp.float32).at[:, :64].set(b0)
    w1p = jnp.zeros((D, D), jnp.float32).at[:64, :32].set(w1)
    b1p = jnp.zeros((1, D), jnp.float32).at[:, :32].set(b1)
    w2p = jnp.zeros((D, D), jnp.float32).at[:32, :2].set(w2)
    b2p = jnp.zeros((1, D), jnp.float32).at[:, :2].set(b2)
    return pl.pallas_call(
        _layer_readout_body,
        grid=(2, GRID),
        in_specs=[spec_r, spec_r, spec_r, spec_r, spec_r, spec_n, spec_b,
                  spec_wp, spec_b, spec_b, spec_b,
                  spec_w, spec_b, spec_w, spec_b, spec_w, spec_b],
        out_specs=spec_r,
        out_shape=jax.ShapeDtypeStruct((N, D), jnp.float32),
        scratch_shapes=[pltpu.VMEM((N, D), jnp.float32),
                        pltpu.VMEM((8, D), jnp.float32)],
    )(h, b, sm, mm, hr, nodew, bpre.reshape(1, D), wp, bpost.reshape(1, D),
      gam.reshape(1, D), bet.reshape(1, D), w0p, b0p, w1p, b1p, w2p, b2p)


_SC_MESH = None


def _sc_mesh():
    global _SC_MESH
    if _SC_MESH is None:
        _SC_MESH = plsc.VectorSubcoreMesh(core_axis_name="c", subcore_axis_name="s",
                                          num_cores=NC, num_subcores=NS)
    return _SC_MESH


def _sc_body(a_hbm, src_hbm, dst_hbm, w_hbm, st_hbm,
             s_hbm, m_hbm, hr_hbm, nsc_hbm,
             st_s, dsm, wsm, st_v, src_v, dst_v, w_v, rows_v,
             stS, stM, stH, stN,
             sem_i1, sem_i2, sem_i3, sem_g):
    wid = lax.axis_index("s") * NC + lax.axis_index("c")
    d_base = wid * DPT
    pltpu.sync_copy(st_hbm.at[wid], st_v)
    stvec = st_v[...]
    for k in range(NSUB + 1):
        st_s[k] = stvec[k]

    zero16 = jnp.zeros((16,), jnp.float32)
    one16 = jnp.ones((16,), jnp.float32)
    neg16 = jnp.full((16,), NEG, jnp.float32)

    def zero_accs():
        return ([zero16] * 8, [neg16] * 8, [zero16] * 8, [zero16] * 3)

    def sub_body(j, _):
        sub_base = d_base + j * SUB

        def zrow(r, c_):
            base = r * D
            for c in range(8):
                stS[pl.ds(base + c * 16, 16)] = zero16
                stH[pl.ds(base + c * 16, 16)] = zero16
            for q in range(3):
                stN[pl.ds(q * SUB * 16 + r * 16, 16)] = zero16
            return c_

        lax.fori_loop(0, SUB, zrow, 0)

        s0 = st_s[j]
        s1 = st_s[j + 1]
        a0 = (s0 // 8) * 8
        nch = (s1 - a0 + CHUNK - 1) // CHUNK

        def flush(cur_ld, accS, accM, accH, accN):
            @pl.when(jnp.logical_and(cur_ld >= 0, cur_ld < SUB))
            def _():
                base = cur_ld * D
                for c in range(8):
                    stS[pl.ds(base + c * 16, 16)] = accS[c]
                    stM[pl.ds(base + c * 16, 16)] = accM[c]
                    stH[pl.ds(base + c * 16, 16)] = accH[c]
                for q in range(3):
                    stN[pl.ds(q * SUB * 16 + cur_ld * 16, 16)] = accN[q]

        def issue_idx(k):
            b = lax.rem(k, 3)
            cb = a0 + jnp.minimum(k, jnp.maximum(nch, 1) - 1) * CHUNK
            dsl = pl.ds(cb, CHUNK)
            pltpu.async_copy(src_hbm.at[dsl], src_v.at[b], sem_i1)
            pltpu.async_copy(dst_hbm.at[dsl], dst_v.at[b], sem_i2)
            pltpu.async_copy(w_hbm.at[dsl], w_v.at[b], sem_i3)

        def wait_idx():
            dsl = pl.ds(0, CHUNK)
            pltpu.make_async_copy(src_hbm.at[dsl], src_v.at[0], sem_i1).wait()
            pltpu.make_async_copy(dst_hbm.at[dsl], dst_v.at[0], sem_i2).wait()
            pltpu.make_async_copy(w_hbm.at[dsl], w_v.at[0], sem_i3).wait()

        def issue_gather(k):
            b = lax.rem(k, 3)
            rb = lax.rem(k, 2)
            pltpu.async_copy(a_hbm.at[src_v.at[b]], rows_v.at[rb], sem_g)

        def wait_gather():
            pltpu.make_async_copy(a_hbm.at[src_v.at[0]], rows_v.at[0],
                                  sem_g).wait()

        def chunk_body(k, carry):
            b = lax.rem(k, 3)
            rb = lax.rem(k, 2)
            wait_gather()
            wait_idx()
            issue_gather(k + 1)
            issue_idx(k + 2)

            def extr(g, c_):
                dst16 = dst_v[b, pl.ds(g * 16, 16)]
                w16 = w_v[b, pl.ds(g * 16, 16)]
                for lane in range(16):
                    dsm[g * 16 + lane] = dst16[lane]
                    wsm[g * 16 + lane] = w16[lane]
                return c_

            lax.fori_loop(0, CHUNK // 16, extr, 0)

            def edge(i, ec):
                cur_ld, accS, accM, accH, accN = ec
                ld = dsm[i] - sub_base
                new_seg = ld != cur_ld

                def do_flush(accs, _cur=cur_ld):
                    flush(_cur, *accs)
                    return zero_accs()

                accS, accM, accH, accN = lax.cond(
                    new_seg, do_flush, lambda accs: accs,
                    (accS, accM, accH, accN))

                bw = jnp.full((16,), wsm[i], jnp.float32)
                accN = [accN[0] + one16, accN[1] + bw,
                        accN[2] + jnp.maximum(bw, -bw)]
                accS = list(accS)
                accM = list(accM)
                accH = list(accH)
                for c in range(8):
                    a = rows_v[rb, i, pl.ds(c * 16, 16)]
                    accS[c] = accS[c] + a
                    accM[c] = jnp.maximum(accM[c], a)
                    accH[c] = accH[c] + bw * a
                return (ld, accS, accM, accH, accN)

            return lax.fori_loop(0, CHUNK, edge, carry)

        carry0 = (jnp.int32(-1),) + zero_accs()
        issue_idx(0)
        wait_idx()
        issue_gather(0)
        issue_idx(1)
        cur_ld, accS, accM, accH, accN = lax.fori_loop(0, nch, chunk_body, carry0)
        wait_gather()
        wait_idx()
        flush(cur_ld, accS, accM, accH, accN)

        pltpu.sync_copy(stS, s_hbm.at[pl.ds(sub_base * D, SUB * D)])
        pltpu.sync_copy(stM, m_hbm.at[pl.ds(sub_base * D, SUB * D)])
        pltpu.sync_copy(stH, hr_hbm.at[pl.ds(sub_base * D, SUB * D)])
        for q in range(3):
            pltpu.sync_copy(
                stN.at[pl.ds(q * SUB * 16, SUB * 16)],
                nsc_hbm.at[pl.ds(q * NPAD * 16 + sub_base * 16, SUB * 16)])
        return _

    lax.fori_loop(0, NSUB, sub_body, 0)


def _sc_segment_call(a, srcp, dstp, wpad, st2d):
    f = pl.kernel(
        _sc_body,
        out_type=[jax.ShapeDtypeStruct((NPAD * D,), jnp.float32),
                  jax.ShapeDtypeStruct((NPAD * D,), jnp.float32),
                  jax.ShapeDtypeStruct((NPAD * D,), jnp.float32),
                  jax.ShapeDtypeStruct((3 * NPAD * 16,), jnp.float32)],
        mesh=_sc_mesh(),
        scratch_types=[
            pltpu.SMEM((16,), jnp.int32),
            pltpu.SMEM((CHUNK,), jnp.int32),
            pltpu.SMEM((CHUNK,), jnp.float32),
            pltpu.VMEM((16,), jnp.int32),
            pltpu.VMEM((3, CHUNK), jnp.int32),
            pltpu.VMEM((3, CHUNK), jnp.int32),
            pltpu.VMEM((3, CHUNK), jnp.float32),
            pltpu.VMEM((2, CHUNK, D), jnp.float32),
            pltpu.VMEM((SUB * D,), jnp.float32),
            pltpu.VMEM((SUB * D,), jnp.float32),
            pltpu.VMEM((SUB * D,), jnp.float32),
            pltpu.VMEM((3 * SUB * 16,), jnp.float32),
            pltpu.SemaphoreType.DMA,
            pltpu.SemaphoreType.DMA,
            pltpu.SemaphoreType.DMA,
            pltpu.SemaphoreType.DMA,
        ],
    )
    s, m, hr, nsc = f(a, srcp, dstp, wpad, st2d)
    return (s.reshape(NPAD, D), m.reshape(NPAD, D), hr.reshape(NPAD, D),
            nsc.reshape(3, NPAD, 16))


def kernel(x, edge_index, eig, snorm_n, params):
    src = edge_index[0]
    dst = edge_index[1]
    w = eig[:, EIG_IDX]

    key, w_s = jax.lax.sort((dst * 16384 + src, w), num_keys=1,
                            is_stable=False)
    dst_s = key // 16384
    src_s = key - dst_s * 16384
    srcp = jnp.concatenate([src_s, jnp.zeros((CHUNK,), jnp.int32)])
    dstp = jnp.concatenate([dst_s, jnp.full((CHUNK,), NPAD, jnp.int32)])
    wpad = jnp.concatenate([w_s, jnp.zeros((CHUNK,), jnp.float32)])
    bounds = jnp.searchsorted(
        dst_s, (jnp.arange(NW * NSUB + 1) * SUB).astype(jnp.int32)).astype(jnp.int32)
    bidx = jnp.minimum(jnp.arange(NW)[:, None] * NSUB + jnp.arange(16)[None, :],
                       NW * NSUB)
    st2d = bounds[bidx].astype(jnp.int32)

    lp0 = params['layers'][0]
    h, a, b = _embed_call(x, params['W_embed'], params['b_embed'],
                          lp0['W_pre'][:D], lp0['W_pre'][D:])

    n_layers = len(params['layers'])
    nodew = None
    for li, lp in enumerate(params['layers']):
        s, m, hr, nsc = _sc_segment_call(a, srcp, dstp, wpad, st2d)
        s, m, hr = s[:N], m[:N], hr[:N]
        if nodew is None:
            nodew = jnp.stack([nsc[0, :N, 0], nsc[1, :N, 0],
                               nsc[2, :N, 0], snorm_n[:, 0]], axis=1)
        wp = jnp.stack([lp['W_post'][0:D], lp['W_post'][D:2 * D],
                        lp['W_post'][2 * D:3 * D], lp['W_post'][3 * D:4 * D],
                        lp['W_post'][4 * D:5 * D]], axis=0)
        if li + 1 < n_layers:
            lpn = params['layers'][li + 1]
            h, a, b = _layer_call(h, b, s, m, hr, nodew, lp['b_pre'], wp,
                                  lp['b_post'], lp['gamma'], lp['beta'],
                                  lpn['W_pre'][:D], lpn['W_pre'][D:])
        else:
            y = _layer_readout_call(h, b, s, m, hr, nodew, lp['b_pre'], wp,
                                    lp['b_post'], lp['gamma'], lp['beta'],
                                    params['readout'])
    return y[:, :2]
